# baseline (speedup 1.0000x reference)
import functools

import jax
import jax.numpy as jnp
from jax import lax
from jax.experimental import pallas as pl
from jax.experimental.pallas import tpu as pltpu
from jax.experimental.pallas import tpu_sc as plsc

_B = 32
_N = 3 * 512 * 512
_K = _N // 8
_NB = 32768
_CHUNK = 8192
_NCH = _N // _CHUNK
_NS = 16
_NSUP = _NB // 256

_mesh = plsc.VectorSubcoreMesh(core_axis_name="c", subcore_axis_name="s")


@functools.partial(
    pl.kernel,
    out_type=jax.ShapeDtypeStruct((_B * 16,), jnp.float32),
    mesh=_mesh,
    compiler_params=pltpu.CompilerParams(
        needs_layout_passes=False, use_tc_tiling_on_sc=True),
    scratch_types=[
        pltpu.VMEM((16, 512), jnp.float32),
        pltpu.VMEM((16, 512), jnp.float32),
        pltpu.VMEM((16, 512), jnp.float32),
        pltpu.VMEM((16, 512), jnp.float32),
        pltpu.VMEM((_NB + 16,), jnp.int32),
        pltpu.VMEM((_NB + 16,), jnp.float32),
        pltpu.VMEM((16,), jnp.float32),
        pltpu.SemaphoreType.DMA,
        pltpu.SemaphoreType.DMA,
        pltpu.SemaphoreType.DMA,
        pltpu.SemaphoreType.DMA,
    ],
)
def _topk_mean_sc(o_hbm, t_hbm, out_hbm, obuf0, obuf1, tbuf0, tbuf1,
                  cnt, sm, rbuf, so0, so1, st0, st1):
    wid = lax.axis_index("c") * _NS + lax.axis_index("s")
    obufs, tbufs = (obuf0, obuf1), (tbuf0, tbuf1)
    osems, tsems = (so0, so1), (st0, st1)

    def in_slice(hbm, c):
        return hbm.at[wid, c // 32, pl.ds((c % 32) * 16, 16), :]

    zi = jnp.zeros((16,), jnp.int32)
    zf = jnp.zeros((16,), jnp.float32)
    ones = jnp.ones((16,), jnp.int32)
    iota = lax.iota(jnp.int32, 16)

    @plsc.parallel_loop(0, _NB, 16, unroll=8)
    def _(i):
        cnt[pl.ds(i, 16)] = zi
        sm[pl.ds(i, 16)] = zf

    for b in range(2):
        pltpu.async_copy(in_slice(o_hbm, jnp.int32(b)), obufs[b], osems[b])
        pltpu.async_copy(in_slice(t_hbm, jnp.int32(b)), tbufs[b], tsems[b])

    def outer_body(g, carry):
        for b in range(2):
            c = g * 2 + b
            pltpu.make_async_copy(
                in_slice(o_hbm, c), obufs[b], osems[b]).wait()
            pltpu.make_async_copy(
                in_slice(t_hbm, c), tbufs[b], tsems[b]).wait()
            ob, tb = obufs[b], tbufs[b]

            @plsc.parallel_loop(0, _CHUNK, 16, unroll=8,
                                carry=(iota + _NB, zf))
            def _(v, pcarry):
                pidx, pval = pcarry
                o = ob[v >> 9, pl.ds(v & 511, 16)]
                t = tb[v >> 9, pl.ds(v & 511, 16)]
                d = o - t
                l = d * d
                bits = lax.bitcast_convert_type(l, jnp.int32)
                idx = lax.shift_right_logical(bits, 16)
                plsc.addupdate_scatter(cnt, [pidx], ones)
                plsc.addupdate_scatter(sm, [pidx], pval)
                return (idx, l)

            lidx, lval = _
            plsc.addupdate_scatter(cnt, [lidx], ones)
            plsc.addupdate_scatter(sm, [lidx], lval)

            nxt = c + 2

            @pl.when(nxt < _NCH)
            def _():
                pltpu.async_copy(in_slice(o_hbm, nxt), obufs[b], osems[b])
                pltpu.async_copy(in_slice(t_hbm, nxt), tbufs[b], tsems[b])
        return carry

    lax.fori_loop(0, _NCH // 2, outer_body, 0)

    def chunk_fine(args):
        pos, s_run, ssum, c_vec, s_vec = args
        sufc = lax.rev(plsc.cumsum(lax.rev(c_vec, (0,))), (0,))
        sufs = lax.rev(plsc.cumsum(lax.rev(s_vec, (0,))), (0,))
        mask = (s_run + sufc) >= _K
        i_star = jnp.max(jnp.where(mask, iota, -1))
        sel = iota == i_star
        suf_at = jnp.sum(jnp.where(sel, sufc, 0))
        c_at = jnp.sum(jnp.where(sel, c_vec, 0))
        sufs_at = jnp.sum(jnp.where(sel, sufs, 0.0))
        s_at = jnp.sum(jnp.where(sel, s_vec, 0.0))
        return (pos * 16 + i_star, s_run + suf_at - c_at,
                ssum + sufs_at - s_at, c_at, s_at, jnp.int32(1))

    def super_fine(args):
        base, s_run0, ssum0, res0 = args

        def mid_body(t, mcarry):
            s_run, ssum, res = mcarry
            pos = base // 16 + 15 - t
            c_vec = cnt[pl.ds(pos * 16, 16)]
            s_vec = sm[pl.ds(pos * 16, 16)]
            tot_c = jnp.sum(c_vec)
            tot_s = jnp.sum(s_vec)
            enter = (res[5] == 0) & ((s_run + tot_c) >= _K)
            res = lax.cond(enter, chunk_fine, lambda a: res,
                           (pos, s_run, ssum, c_vec, s_vec))
            return (s_run + tot_c, ssum + tot_s, res)

        _, _, res = lax.fori_loop(0, 16, mid_body, (s_run0, ssum0, res0))
        return res

    def super_body(g, carry):
        s_run, ssum, res = carry
        base = (_NSUP - 1 - g) * 256
        acc_c = zi
        acc_s = zf
        for t in range(16):
            acc_c = acc_c + cnt[pl.ds(base + t * 16, 16)]
            acc_s = acc_s + sm[pl.ds(base + t * 16, 16)]
        tot_c = jnp.sum(acc_c)
        tot_s = jnp.sum(acc_s)
        enter = (res[5] == 0) & ((s_run + tot_c) >= _K)
        res = lax.cond(enter, super_fine, lambda a: res,
                       (base, s_run, ssum, res))
        return (s_run + tot_c, ssum + tot_s, res)

    res0 = (jnp.int32(0), jnp.int32(0), jnp.float32(0.0), jnp.int32(1),
            jnp.float32(0.0), jnp.int32(0))
    (_, _, res) = lax.fori_loop(
        0, _NSUP, super_body, (jnp.int32(0), jnp.float32(0.0), res0))
    b_star, cnt_above, sum_above, cnt_b, sum_b, _ = res

    b_v = jnp.zeros((16,), jnp.int32) + b_star
    lo_v = lax.bitcast_convert_type(lax.shift_left(b_v, 16), jnp.float32)
    hi_v = lax.bitcast_convert_type(lax.shift_left(b_v + 1, 16), jnp.float32)
    m_v = jnp.zeros((16,), jnp.float32) + (_K - cnt_above).astype(jnp.float32)
    cb_v = jnp.zeros((16,), jnp.float32) + cnt_b.astype(jnp.float32)
    mu_v = (jnp.zeros((16,), jnp.float32) + sum_b) / cb_v
    per_v = jnp.minimum(mu_v + (1.0 - m_v / cb_v) * (hi_v - lo_v) * 0.5, hi_v)
    row_v = (jnp.zeros((16,), jnp.float32) + sum_above) + m_v * per_v
    rbuf[...] = row_v * (1.0 / (_B * _K))
    pltpu.sync_copy(rbuf, out_hbm.at[pl.ds(wid * 16, 16)])


def kernel(output, target):
    rows = _topk_mean_sc(output, target)
    return jnp.sum(rows.reshape(_B, 16)[:, 0])


# speedup vs baseline: 1.2186x; 1.2186x over previous
import functools

import jax
import jax.numpy as jnp
from jax import lax
from jax.experimental import pallas as pl
from jax.experimental.pallas import tpu as pltpu
from jax.experimental.pallas import tpu_sc as plsc

_B = 32
_N = 3 * 512 * 512
_K = _N // 8
_NB = 32768
_CHUNK = 8192
_NCH = _N // _CHUNK
_NS = 16
_NSUP = _NB // 256
_TINY = 2.0 ** -126

_mesh = plsc.VectorSubcoreMesh(core_axis_name="c", subcore_axis_name="s")


@functools.partial(
    pl.kernel,
    out_type=jax.ShapeDtypeStruct((_B * 16,), jnp.float32),
    mesh=_mesh,
    compiler_params=pltpu.CompilerParams(
        needs_layout_passes=False, use_tc_tiling_on_sc=True),
    scratch_types=[
        pltpu.VMEM((16, 512), jnp.float32),
        pltpu.VMEM((16, 512), jnp.float32),
        pltpu.VMEM((16, 512), jnp.float32),
        pltpu.VMEM((16, 512), jnp.float32),
        pltpu.VMEM((_NB,), jnp.float32),
        pltpu.VMEM((128,), jnp.float32),
        pltpu.VMEM((16,), jnp.float32),
        pltpu.SemaphoreType.DMA,
        pltpu.SemaphoreType.DMA,
        pltpu.SemaphoreType.DMA,
        pltpu.SemaphoreType.DMA,
    ],
)
def _topk_mean_sc(o_hbm, t_hbm, out_hbm, obuf0, obuf1, tbuf0, tbuf1,
                  sm, rcp, rbuf, so0, so1, st0, st1):
    wid = lax.axis_index("c") * _NS + lax.axis_index("s")
    obufs, tbufs = (obuf0, obuf1), (tbuf0, tbuf1)
    osems, tsems = (so0, so1), (st0, st1)

    def in_slice(hbm, c):
        return hbm.at[wid, c // 32, pl.ds((c % 32) * 16, 16), :]

    zf = jnp.zeros((16,), jnp.float32)
    zi = jnp.zeros((16,), jnp.int32)
    iota = lax.iota(jnp.int32, 16)
    iota_f = iota.astype(jnp.float32)

    @plsc.parallel_loop(0, _NB, 16, unroll=8)
    def _(i):
        sm[pl.ds(i, 16)] = zf

    for jj in range(8):
        mid = 1.0 + (jj * 16 + iota_f + 0.5) * (1.0 / 128.0)
        rcp[pl.ds(jj * 16, 16)] = 1.0 / mid

    for b in range(2):
        pltpu.async_copy(in_slice(o_hbm, jnp.int32(b)), obufs[b], osems[b])
        pltpu.async_copy(in_slice(t_hbm, jnp.int32(b)), tbufs[b], tsems[b])

    def outer_body(g, carry):
        for b in range(2):
            c = g * 2 + b
            pltpu.make_async_copy(
                in_slice(o_hbm, c), obufs[b], osems[b]).wait()
            pltpu.make_async_copy(
                in_slice(t_hbm, c), tbufs[b], tsems[b]).wait()
            ob, tb = obufs[b], tbufs[b]

            @plsc.parallel_loop(0, _CHUNK, 16, unroll=8)
            def _(v):
                o = ob[v >> 9, pl.ds(v & 511, 16)]
                t = tb[v >> 9, pl.ds(v & 511, 16)]
                d = o - t
                l = d * d
                bits = lax.bitcast_convert_type(l, jnp.int32)
                idx = lax.shift_right_logical(bits, 16)
                mant = lax.bitcast_convert_type(
                    (bits & 0x7FFFFF) | 0x3F800000, jnp.float32)
                plsc.addupdate_scatter(sm, [idx], mant)

            nxt = c + 2

            @pl.when(nxt < _NCH)
            def _():
                pltpu.async_copy(in_slice(o_hbm, nxt), obufs[b], osems[b])
                pltpu.async_copy(in_slice(t_hbm, nxt), tbufs[b], tsems[b])
        return carry

    lax.fori_loop(0, _NCH // 2, outer_body, 0)

    def chunk_vals(pos):
        a = sm[pl.ds(pos * 16, 16)]
        r = rcp[pl.ds((pos & 7) * 16, 16)]
        chat = a * r
        e = lax.shift_right_logical(pos, 3)
        scale = lax.bitcast_convert_type(
            lax.shift_left(zi + jnp.maximum(e, 1), 23), jnp.float32)
        sums = jnp.where((zi + e) >= 1, a * scale, (a - chat) * _TINY)
        return chat, sums

    kf = jnp.float32(_K)

    def chunk_fine(args):
        pos, s_run, ssum, c_vec, s_vec = args
        sufc = lax.rev(plsc.cumsum(lax.rev(c_vec, (0,))), (0,))
        sufs = lax.rev(plsc.cumsum(lax.rev(s_vec, (0,))), (0,))
        mask = (s_run + sufc) >= kf
        i_star = jnp.max(jnp.where(mask, iota, -1))
        sel = iota == i_star
        suf_at = jnp.sum(jnp.where(sel, sufc, 0.0))
        c_at = jnp.sum(jnp.where(sel, c_vec, 0.0))
        sufs_at = jnp.sum(jnp.where(sel, sufs, 0.0))
        s_at = jnp.sum(jnp.where(sel, s_vec, 0.0))
        return (pos * 16 + i_star, s_run + suf_at - c_at,
                ssum + sufs_at - s_at, c_at, s_at, jnp.int32(1))

    def super_fine(args):
        base_pos, s_run0, ssum0, res0 = args

        def mid_body(t, mcarry):
            s_run, ssum, res = mcarry
            pos = base_pos + 15 - t
            c_vec, s_vec = chunk_vals(pos)
            tot_c = jnp.sum(c_vec)
            tot_s = jnp.sum(s_vec)
            enter = (res[5] == 0) & ((s_run + tot_c) >= kf)
            res = lax.cond(enter, chunk_fine, lambda a: res,
                           (pos, s_run, ssum, c_vec, s_vec))
            return (s_run + tot_c, ssum + tot_s, res)

        _, _, res = lax.fori_loop(0, 16, mid_body, (s_run0, ssum0, res0))
        return res

    def super_body(g, carry):
        s_run, ssum, res = carry
        base_pos = (_NSUP - 1 - g) * 16
        acc_c = zf
        acc_s = zf
        for t in range(16):
            c_vec, s_vec = chunk_vals(base_pos + t)
            acc_c = acc_c + c_vec
            acc_s = acc_s + s_vec
        tot_c = jnp.sum(acc_c)
        tot_s = jnp.sum(acc_s)
        enter = (res[5] == 0) & ((s_run + tot_c) >= kf)
        res = lax.cond(enter, super_fine, lambda a: res,
                       (base_pos, s_run, ssum, res))
        return (s_run + tot_c, ssum + tot_s, res)

    res0 = (jnp.int32(0), jnp.float32(0.0), jnp.float32(0.0),
            jnp.float32(1.0), jnp.float32(0.0), jnp.int32(0))
    (_, _, res) = lax.fori_loop(
        0, _NSUP, super_body, (jnp.float32(0.0), jnp.float32(0.0), res0))
    b_star, cnt_above, sum_above, cnt_b, sum_b, _ = res

    b_v = zi + b_star
    lo_v = lax.bitcast_convert_type(lax.shift_left(b_v, 16), jnp.float32)
    hi_v = lax.bitcast_convert_type(lax.shift_left(b_v + 1, 16), jnp.float32)
    m_v = zf + (kf - cnt_above)
    cb_v = zf + cnt_b
    mu_v = (zf + sum_b) / cb_v
    per_v = jnp.minimum(mu_v + (1.0 - m_v / cb_v) * (hi_v - lo_v) * 0.5, hi_v)
    row_v = (zf + sum_above) + m_v * per_v
    rbuf[...] = row_v * (1.0 / (_B * _K))
    pltpu.sync_copy(rbuf, out_hbm.at[pl.ds(wid * 16, 16)])


def kernel(output, target):
    rows = _topk_mean_sc(output, target)
    return jnp.sum(rows.reshape(_B, 16)[:, 0])
